# baseline (speedup 1.0000x reference)
import functools

import jax
import jax.numpy as jnp
from jax import lax
from jax.experimental import pallas as pl
from jax.experimental.pallas import tpu as pltpu
from jax.experimental.pallas import tpu_sc as plsc

ROWS, COLS = 16384, 200
VOCAB, DIM = 64, 64
NW = 32
I_PER_W = ROWS // NW
IB_PER_W = I_PER_W // 128
JO_TOT = (COLS // 8) * IB_PER_W
L = 16


def _make_kernel():
    mesh = plsc.VectorSubcoreMesh(core_axis_name="c", subcore_axis_name="s")

    @functools.partial(
        pl.kernel,
        mesh=mesh,
        out_type=jax.ShapeDtypeStruct((COLS, DIM, ROWS), jnp.float32),
        scratch_types=[
            pltpu.VMEM((VOCAB * 512,), jnp.float32),
            pltpu.VMEM((8, 128), jnp.int32),
            pltpu.VMEM((8, 128), jnp.int32),
            pltpu.VMEM((DIM, 128), jnp.float32),
            pltpu.VMEM((DIM, 128), jnp.float32),
            pltpu.SemaphoreType.DMA,
            pltpu.SemaphoreType.DMA,
            pltpu.SemaphoreType.DMA,
            pltpu.SemaphoreType.DMA,
        ],
        compiler_params=pltpu.CompilerParams(
            use_tc_tiling_on_sc=True, needs_layout_passes=False),
    )
    def gather_kernel(xt_hbm, tabp_hbm, out_hbm,
                      tab_v, ia, ib_, oa, ob_,
                      sia, sib, soa, sob):
        wid = lax.axis_index("s") * 2 + lax.axis_index("c")
        i_w = wid * I_PER_W
        idx_v = (ia, ib_)
        out_v = (oa, ob_)
        sem_i = (sia, sib)
        sem_o = (soa, sob)

        pltpu.sync_copy(tabp_hbm, tab_v)

        def coords(step):
            ib = step // 25
            jo = step - ib * 25
            return jo * 8, i_w + ib * 128

        for b in range(2):
            j0, i0 = coords(b)
            pltpu.async_copy(
                xt_hbm.at[pl.ds(j0, 8), pl.ds(i0, 128)], idx_v[b], sem_i[b])

        def jot_body(t, carry):
            for b in range(2):
                step = 2 * t + b
                j0, i0 = coords(step)
                pltpu.make_async_copy(
                    xt_hbm.at[pl.ds(0, 8), pl.ds(0, 128)], idx_v[b],
                    sem_i[b]).wait()

                def jj_body(jp, carry_j):
                    for par in range(2):
                        jj = 2 * jp + par
                        gstep = step * 8 + jj
                        @pl.when(gstep >= 2)
                        def _():
                            pltpu.make_async_copy(
                                out_v[par],
                                out_hbm.at[0, :, pl.ds(0, 128)],
                                sem_o[par]).wait()

                        cols = [idx_v[b][jj, pl.ds(c * L, L)] * 8
                                for c in range(8)]
                        for k in range(DIM):
                            gs = [plsc.load_gather(
                                      tab_v, [cols[c] + (k * 512)])
                                  for c in range(8)]
                            for c in range(8):
                                out_v[par][k, pl.ds(c * L, L)] = gs[c]

                        pltpu.async_copy(
                            out_v[par],
                            out_hbm.at[j0 + jj, :, pl.ds(i0, 128)],
                            sem_o[par])
                    return carry_j

                lax.fori_loop(0, 4, jj_body, 0)

                @pl.when(step < JO_TOT - 2)
                def _():
                    j2, i2 = coords(step + 2)
                    pltpu.async_copy(
                        xt_hbm.at[pl.ds(j2, 8), pl.ds(i2, 128)],
                        idx_v[b], sem_i[b])
            return 0

        lax.fori_loop(0, JO_TOT // 2, jot_body, 0)

        for par in range(2):
            pltpu.make_async_copy(
                out_v[par], out_hbm.at[0, :, pl.ds(0, 128)],
                sem_o[par]).wait()

    return gather_kernel


_gather = _make_kernel()


@jax.jit
def kernel(X, table):
    xt = X.T
    tabp = (jnp.zeros((DIM, 512), jnp.float32)
            .at[:, ::8].set(table.T).reshape(DIM * 512))
    out3 = _gather(xt, tabp)
    return jnp.transpose(out3, (2, 0, 1))


# speedup vs baseline: 1.2273x; 1.2273x over previous
import functools

import jax
import jax.numpy as jnp
from jax import lax
from jax.experimental import pallas as pl
from jax.experimental.pallas import tpu as pltpu
from jax.experimental.pallas import tpu_sc as plsc

ROWS, COLS = 16384, 200
VOCAB, DIM = 64, 64
NW = 32
I_PER_W = ROWS // NW
IB_PER_W = I_PER_W // 128
JO_TOT = (COLS // 8) * IB_PER_W
L = 16


def _make_kernel():
    mesh = plsc.VectorSubcoreMesh(core_axis_name="c", subcore_axis_name="s")

    @functools.partial(
        pl.kernel,
        mesh=mesh,
        out_type=jax.ShapeDtypeStruct((COLS, DIM, ROWS), jnp.float32),
        scratch_types=[
            pltpu.VMEM((VOCAB, 128), jnp.float32),
            pltpu.VMEM((8, 128), jnp.int32),
            pltpu.VMEM((8, 128), jnp.int32),
            pltpu.VMEM((DIM, 128), jnp.float32),
            pltpu.VMEM((DIM, 128), jnp.float32),
            pltpu.SemaphoreType.DMA,
            pltpu.SemaphoreType.DMA,
            pltpu.SemaphoreType.DMA,
            pltpu.SemaphoreType.DMA,
        ],
        compiler_params=pltpu.CompilerParams(
            use_tc_tiling_on_sc=True, needs_layout_passes=False),
    )
    def gather_kernel(xt_hbm, tabp_hbm, out_hbm,
                      tab_v, ia, ib_, oa, ob_,
                      sia, sib, soa, sob):
        wid = lax.axis_index("s") * 2 + lax.axis_index("c")
        i_w = wid * I_PER_W
        idx_v = (ia, ib_)
        out_v = (oa, ob_)
        sem_i = (sia, sib)
        sem_o = (soa, sob)

        pltpu.sync_copy(tabp_hbm, tab_v)

        def coords(step):
            ib = step // 25
            jo = step - ib * 25
            return jo * 8, i_w + ib * 128

        for b in range(2):
            j0, i0 = coords(b)
            pltpu.async_copy(
                xt_hbm.at[pl.ds(j0, 8), pl.ds(i0, 128)], idx_v[b], sem_i[b])

        def jot_body(t, carry):
            for b in range(2):
                step = 2 * t + b
                j0, i0 = coords(step)
                pltpu.make_async_copy(
                    xt_hbm.at[pl.ds(0, 8), pl.ds(0, 128)], idx_v[b],
                    sem_i[b]).wait()

                def jj_body(jp, carry_j):
                    for par in range(2):
                        jj = 2 * jp + par
                        gstep = step * 8 + jj
                        @pl.when(gstep >= 2)
                        def _():
                            pltpu.make_async_copy(
                                out_v[par],
                                out_hbm.at[0, :, pl.ds(0, 128)],
                                sem_o[par]).wait()

                        cols = [idx_v[b][jj, pl.ds(c * L, L)] * 2
                                for c in range(8)]
                        for k2 in range(DIM // 2):
                            gs = []
                            for kk in range(2):
                                k = 2 * k2 + kk
                                krow = jnp.full((L,), k, jnp.int32)
                                gs += [plsc.load_gather(
                                           tab_v, [krow, cols[c]])
                                       for c in range(8)]
                            for kk in range(2):
                                k = 2 * k2 + kk
                                for c in range(8):
                                    out_v[par][k, pl.ds(c * L, L)] = (
                                        gs[kk * 8 + c])

                        pltpu.async_copy(
                            out_v[par],
                            out_hbm.at[j0 + jj, :, pl.ds(i0, 128)],
                            sem_o[par])
                    return carry_j

                lax.fori_loop(0, 4, jj_body, 0)

                @pl.when(step < JO_TOT - 2)
                def _():
                    j2, i2 = coords(step + 2)
                    pltpu.async_copy(
                        xt_hbm.at[pl.ds(j2, 8), pl.ds(i2, 128)],
                        idx_v[b], sem_i[b])
            return 0

        lax.fori_loop(0, JO_TOT // 2, jot_body, 0)

        for par in range(2):
            pltpu.make_async_copy(
                out_v[par], out_hbm.at[0, :, pl.ds(0, 128)],
                sem_o[par]).wait()

    return gather_kernel


_gather = _make_kernel()


@jax.jit
def kernel(X, table):
    xt = X.T
    tabp = jnp.zeros((DIM, 128), jnp.float32).at[:, ::2].set(table.T)
    out3 = _gather(xt, tabp)
    return jnp.transpose(out3, (2, 0, 1))


# speedup vs baseline: 1.2738x; 1.0379x over previous
import functools

import jax
import jax.numpy as jnp
from jax import lax
from jax.experimental import pallas as pl
from jax.experimental.pallas import tpu as pltpu
from jax.experimental.pallas import tpu_sc as plsc

ROWS, COLS = 16384, 200
VOCAB, DIM = 64, 64
NW = 32
I_PER_W = ROWS // NW
IB_PER_W = I_PER_W // 128
JO_TOT = (COLS // 8) * IB_PER_W
L = 16


def _make_kernel():
    mesh = plsc.VectorSubcoreMesh(core_axis_name="c", subcore_axis_name="s")

    @functools.partial(
        pl.kernel,
        mesh=mesh,
        out_type=jax.ShapeDtypeStruct((COLS, DIM, ROWS), jnp.float32),
        scratch_types=[
            pltpu.VMEM((VOCAB, 128), jnp.float32),
            pltpu.VMEM((8, 128), jnp.int32),
            pltpu.VMEM((8, 128), jnp.int32),
            pltpu.VMEM((DIM, 128), jnp.float32),
            pltpu.VMEM((DIM, 128), jnp.float32),
            pltpu.SemaphoreType.DMA,
            pltpu.SemaphoreType.DMA,
            pltpu.SemaphoreType.DMA,
            pltpu.SemaphoreType.DMA,
        ],
        compiler_params=pltpu.CompilerParams(
            use_tc_tiling_on_sc=True, needs_layout_passes=False),
    )
    def gather_kernel(xt_hbm, tabp_hbm, out_hbm,
                      tab_v, ia, ib_, oa, ob_,
                      sia, sib, soa, sob):
        wid = lax.axis_index("s") * 2 + lax.axis_index("c")
        i_w = wid * I_PER_W
        idx_v = (ia, ib_)
        out_v = (oa, ob_)
        sem_i = (sia, sib)
        sem_o = (soa, sob)

        pltpu.sync_copy(tabp_hbm, tab_v)

        def coords(step):
            ib = step // 25
            jo = step - ib * 25
            return jo * 8, i_w + ib * 128

        for b in range(2):
            j0, i0 = coords(b)
            pltpu.async_copy(
                xt_hbm.at[pl.ds(j0, 8), pl.ds(i0, 128)], idx_v[b], sem_i[b])

        def jot_body(t, carry):
            for b in range(2):
                step = 2 * t + b
                j0, i0 = coords(step)
                pltpu.make_async_copy(
                    xt_hbm.at[pl.ds(0, 8), pl.ds(0, 128)], idx_v[b],
                    sem_i[b]).wait()

                def jj_body(jp, carry_j):
                    for par in range(2):
                        jj = 2 * jp + par
                        gstep = step * 8 + jj
                        @pl.when(gstep >= 2)
                        def _():
                            pltpu.make_async_copy(
                                out_v[par],
                                out_hbm.at[0, :, pl.ds(0, 128)],
                                sem_o[par]).wait()

                        cols = [idx_v[b][jj, pl.ds(c * L, L)] * 2
                                for c in range(8)]
                        for k in range(DIM):
                            krow = jnp.full((L,), k, jnp.int32)
                            gs = [plsc.load_gather(tab_v, [krow, cols[c]])
                                  for c in range(8)]
                            for c in range(8):
                                out_v[par][k, pl.ds(c * L, L)] = gs[c]

                        pltpu.async_copy(
                            out_v[par],
                            out_hbm.at[j0 + jj, :, pl.ds(i0, 128)],
                            sem_o[par])
                    return carry_j

                lax.fori_loop(0, 4, jj_body, 0)

                @pl.when(step < JO_TOT - 2)
                def _():
                    j2, i2 = coords(step + 2)
                    pltpu.async_copy(
                        xt_hbm.at[pl.ds(j2, 8), pl.ds(i2, 128)],
                        idx_v[b], sem_i[b])
            return 0

        lax.fori_loop(0, JO_TOT // 2, jot_body, 0)

        for par in range(2):
            pltpu.make_async_copy(
                out_v[par], out_hbm.at[0, :, pl.ds(0, 128)],
                sem_o[par]).wait()

    return gather_kernel


_gather = _make_kernel()


@jax.jit
def kernel(X, table):
    xt = X.T
    tabp = jnp.zeros((DIM, 128), jnp.float32).at[:, ::2].set(table.T)
    out3 = _gather(xt, tabp)
    return jnp.transpose(out3, (2, 0, 1))
